# baseline (speedup 1.0000x reference)
import functools

import jax
import jax.numpy as jnp
from jax import lax
from jax.experimental import pallas as pl
from jax.experimental.pallas import tpu as pltpu
from jax.experimental.pallas import tpu_sc as plsc

B = 16384
L = 50
D = 64
VOCAB = 1000000
LANES = 16
ND = D // LANES
NC = 2
NS = 16
NW = NC * NS
BPW = B // NW

DCHUNK = 96
NDETILE = 327

PAIRS = 1
CB = 8
ROWS_PER_CHUNK = CB * PAIRS
LUNROLL = 10


def _make_kernel():
    mesh = plsc.VectorSubcoreMesh(core_axis_name="c", subcore_axis_name="s")

    @functools.partial(
        pl.kernel,
        mesh=mesh,
        out_type=jax.ShapeDtypeStruct((B, D), jnp.float32),
        scratch_types=[
            pltpu.MemorySpace.HBM((VOCAB, D), jnp.float32),
            pltpu.MemorySpace.VMEM_SHARED((NS, 2, DCHUNK, D), jnp.float32),
            pltpu.MemorySpace.VMEM((2, CB, PAIRS * L), jnp.int32),
            pltpu.MemorySpace.VMEM((2, CB, PAIRS * L, D), jnp.float32),
            pltpu.MemorySpace.VMEM((2, ROWS_PER_CHUNK, D), jnp.float32),
            pltpu.SemaphoreType.REGULAR,
            pltpu.SemaphoreType.DMA,
            pltpu.SemaphoreType.DMA,
            pltpu.SemaphoreType.DMA,
            pltpu.SemaphoreType.DMA,
            pltpu.SemaphoreType.DMA,
            pltpu.SemaphoreType.DMA,
        ],
    )
    def emb_mean(labels_hbm, table_hbm, out_hbm, scratch,
                 detbuf, idx_v, rows_v, out_v, xsem,
                 sa0, sa1, sb0, sb1, sc0, sc1):
        sid = lax.axis_index("s")
        core = lax.axis_index("c")
        wid = sid * NC + core
        sem_a = (sa0, sa1)
        sem_b = (sb0, sb1)
        sem_c = (sc0, sc1)

        start = (VOCAB * wid // (NW * 16)) * 16
        end = (VOCAB * (wid + 1) // (NW * 16)) * 16
        last_start = end - DCHUNK

        def chunk_start(c):
            return jnp.minimum(start + c * DCHUNK, last_start)

        def in_copy(c, p):
            return pltpu.make_async_copy(
                table_hbm.at[pl.ds(pl.multiple_of(chunk_start(c), 16),
                                   DCHUNK)],
                detbuf.at[sid].at[p], sem_a[p])

        def out_copy(c, p):
            return pltpu.make_async_copy(
                detbuf.at[sid].at[p],
                scratch.at[pl.ds(pl.multiple_of(chunk_start(c), 16),
                                 DCHUNK)],
                sem_b[p])

        in_copy(0, 0).start()
        in_copy(1, 1).start()

        def det_body(c2, carry):
            for p in (0, 1):
                c = c2 * 2 + p
                in_copy(c, p).wait()
                out_copy(c, p).start()
                out_copy(c, p).wait()
                in_copy(c + 2, p).start()
            return carry

        lax.fori_loop(0, (NDETILE - 3) // 2, det_body, 0)
        c0 = NDETILE - 3
        in_copy(c0, 0).wait()
        out_copy(c0, 0).start()
        in_copy(c0 + 1, 1).wait()
        out_copy(c0 + 1, 1).start()
        out_copy(c0, 0).wait()
        in_copy(c0 + 2, 0).start()
        in_copy(c0 + 2, 0).wait()
        out_copy(c0 + 2, 0).start()
        out_copy(c0 + 1, 1).wait()
        out_copy(c0 + 2, 0).wait()

        plsc.subcore_barrier()
        pltpu.semaphore_signal(xsem, 1, core_index=0)

        @pl.when(core == 0)
        def _gather_phase():
            pltpu.semaphore_wait(xsem, 2)

            bpw2 = BPW * NC
            nchunk2 = bpw2 // ROWS_PER_CHUNK
            pair_base = sid * (bpw2 // PAIRS)

            def idx_copy(c, p):
                return pltpu.make_async_copy(
                    labels_hbm.at[pl.ds(
                        pl.multiple_of(pair_base + c * CB, 8), CB)],
                    idx_v.at[p], sem_a[p])

            def gather_copy(p, i):
                return pltpu.make_async_copy(
                    scratch.at[idx_v.at[p].at[i]],
                    rows_v.at[p].at[i], sem_b[p])

            def fire_gathers(p):
                for i in range(CB):
                    gather_copy(p, i).start()

            def wait_gathers(p):
                for i in range(CB):
                    gather_copy(p, i).wait()

            def res_copy(c, p):
                return pltpu.make_async_copy(
                    out_v.at[p],
                    out_hbm.at[pl.ds(pl.multiple_of(
                        sid * bpw2 + c * ROWS_PER_CHUNK, 8),
                        ROWS_PER_CHUNK)],
                    sem_c[p])

            def accumulate(c, p):
                for i in range(CB):
                    for h in range(PAIRS):
                        jbase = h * L

                        def l_body(j, acc, i=i, jbase=jbase):
                            new = []
                            for d in range(ND):
                                a = acc[d]
                                for u in range(LUNROLL):
                                    a = a + rows_v[
                                        p, i, jbase + j * LUNROLL + u,
                                        pl.ds(d * LANES, LANES)]
                                new.append(a)
                            return tuple(new)

                        acc0 = tuple(jnp.zeros((LANES,), jnp.float32)
                                     for _ in range(ND))
                        acc = lax.fori_loop(0, L // LUNROLL, l_body, acc0)
                        r = i * PAIRS + h
                        for d in range(ND):
                            out_v[p, r, pl.ds(d * LANES, LANES)] = (
                                acc[d] * (1.0 / L))

            def step(c, p, prefetch, wait_res):
                idx_copy(c + 1, 1 - p).wait()
                fire_gathers(1 - p)
                wait_gathers(p)
                if prefetch:
                    idx_copy(c + 2, p).start()
                if wait_res:
                    res_copy(c, p).wait()
                accumulate(c, p)
                res_copy(c, p).start()

            idx_copy(0, 0).start()
            idx_copy(1, 1).start()
            idx_copy(0, 0).wait()
            fire_gathers(0)

            step(0, 0, prefetch=True, wait_res=False)
            step(1, 1, prefetch=True, wait_res=False)

            def chunk_pair(c2, carry):
                for p in (0, 1):
                    c = c2 * 2 + p
                    step(c, p, prefetch=True, wait_res=True)
                return carry

            lax.fori_loop(1, nchunk2 // 2 - 1, chunk_pair, 0)

            c0 = nchunk2 - 2
            idx_copy(c0 + 1, 1).wait()
            fire_gathers(1)
            wait_gathers(0)
            res_copy(c0, 0).wait()
            accumulate(c0, 0)
            res_copy(c0, 0).start()
            wait_gathers(1)
            res_copy(c0 + 1, 1).wait()
            accumulate(c0 + 1, 1)
            res_copy(c0 + 1, 1).start()
            res_copy(c0, 0).wait()
            res_copy(c0 + 1, 1).wait()

    return emb_mean


_emb_mean = _make_kernel()


@jax.jit
def kernel(label_text, table):
    return _emb_mean(label_text, table)


# speedup vs baseline: 1.2800x; 1.2800x over previous
import functools

import jax
import jax.numpy as jnp
from jax import lax
from jax.experimental import pallas as pl
from jax.experimental.pallas import tpu as pltpu
from jax.experimental.pallas import tpu_sc as plsc

B = 16384
L = 50
D = 64
LANES = 16
ND = D // LANES
NC = 2
NS = 16
NW = NC * NS
BPW = B // NW
PAIRS = 2
CB = 4
ROWS_PER_CHUNK = CB * PAIRS
NCHUNK = BPW // ROWS_PER_CHUNK
LUNROLL = 10


def _make_kernel():
    mesh = plsc.VectorSubcoreMesh(core_axis_name="c", subcore_axis_name="s")

    @functools.partial(
        pl.kernel,
        mesh=mesh,
        compiler_params=pltpu.CompilerParams(use_tc_tiling_on_sc=False),
        out_type=jax.ShapeDtypeStruct((B, D), jnp.float32),
        scratch_types=[
            pltpu.VMEM((2, CB, PAIRS * L), jnp.int32),
            pltpu.VMEM((2, CB, PAIRS * L, D), jnp.float32),
            pltpu.VMEM((BPW, D), jnp.float32),
            pltpu.SemaphoreType.DMA,
            pltpu.SemaphoreType.DMA,
            pltpu.SemaphoreType.DMA,
            pltpu.SemaphoreType.DMA,
        ],
    )
    def emb_mean(labels_hbm, table_hbm, out_hbm, idx_v, rows_v, out_v,
                 sem_i0, sem_i1, sem_g0, sem_g1):
        wid = lax.axis_index("s") * NC + lax.axis_index("c")
        pair_base = wid * (BPW // PAIRS)
        sem_i = (sem_i0, sem_i1)
        sem_g = (sem_g0, sem_g1)

        def idx_copy(c, p):
            return pltpu.make_async_copy(
                labels_hbm.at[pl.ds(pair_base + c * CB, CB)],
                idx_v.at[p], sem_i[p])

        def gather_copy(p, i):
            return pltpu.make_async_copy(
                table_hbm.at[idx_v.at[p].at[i]],
                rows_v.at[p].at[i], sem_g[p])

        def fire_gathers(p):
            for i in range(CB):
                gather_copy(p, i).start()

        def wait_gathers(p):
            for i in range(CB):
                gather_copy(p, i).wait()

        def accumulate(c, p):
            for i in range(CB):
                for h in range(PAIRS):
                    jbase = h * L

                    def l_body(j, acc, i=i, jbase=jbase):
                        new = []
                        for d in range(ND):
                            a = acc[d]
                            for u in range(LUNROLL):
                                a = a + rows_v[p, i, jbase + j * LUNROLL + u,
                                               pl.ds(d * LANES, LANES)]
                            new.append(a)
                        return tuple(new)

                    acc0 = tuple(
                        jnp.zeros((LANES,), jnp.float32)
                        for _ in range(ND)
                    )
                    acc = lax.fori_loop(0, L // LUNROLL, l_body, acc0)
                    r = (c * CB + i) * PAIRS + h
                    for d in range(ND):
                        out_v[r, pl.ds(d * LANES, LANES)] = (
                            acc[d] * (1.0 / L))

        idx_copy(0, 0).start()
        idx_copy(1, 1).start()
        idx_copy(0, 0).wait()
        fire_gathers(0)

        def chunk_pair(c2, carry):
            for p in (0, 1):
                c = c2 * 2 + p
                idx_copy(c + 1, 1 - p).wait()
                fire_gathers(1 - p)
                wait_gathers(p)
                idx_copy(c + 2, p).start()
                accumulate(c, p)
            return carry

        lax.fori_loop(0, NCHUNK // 2 - 1, chunk_pair, 0)

        idx_copy(NCHUNK - 1, 1).wait()
        fire_gathers(1)
        wait_gathers(0)
        accumulate(NCHUNK - 2, 0)
        wait_gathers(1)
        accumulate(NCHUNK - 1, 1)

        pltpu.sync_copy(
            out_v, out_hbm.at[pl.ds(wid * BPW, BPW)])

    return emb_mean


_emb_mean = _make_kernel()


@jax.jit
def kernel(label_text, table):
    labels2 = label_text.reshape(B // PAIRS, PAIRS * L)
    return _emb_mean(labels2, table)
